# Initial kernel scaffold; baseline (speedup 1.0000x reference)
import jax
import jax.numpy as jnp
from jax.experimental import pallas as pl


def kernel(feat, position, edge_index, W_Q, b_Q, W_K, b_K, W_V, b_V, W_e, b_e, W_s, b_s):
    raise NotImplementedError("write your pallas kernel here")


# speedup vs baseline: 12.3634x; 12.3634x over previous
import functools

import jax
import jax.numpy as jnp
import numpy as np
from jax import lax
from jax.experimental import pallas as pl
from jax.experimental.pallas import tpu as pltpu
from jax.experimental.pallas import tpu_sc as plsc

N = 10000
E = 320000
IN_DIM = 128
OUT_DIM = 16
NUM_HEADS = 8
D = OUT_DIM * NUM_HEADS
EPS = 1e-07

NC = 2
NS = 16
NW = NC * NS
NPT = 320
NPAD = NW * NPT
DUMP = NPT
CH = 32
EPADDED = E + 2 * CH


def _butterfly_sum(x, perms):
  for p in perms:
    x = x + jnp.take_along_axis(x, p, axis=0, mode="promise_in_bounds")
  return x


def _edge_kernel(q_tab, k_tab, v_tab, pe_tab, srcs_h, dsts_h, prm_h, meta_h,
                 wv_out, z_out,
                 src_v, dst_v, prm_v, meta_v, kb, qb, vb, pb, sem):
  w = lax.axis_index("c") * NS + lax.axis_index("s")

  lanes = lax.iota(jnp.int32, 16)
  perms = [lanes ^ k for k in (1, 2, 4, 8)]
  quart = jnp.full((16,), 0.25, jnp.float32)
  zero16 = jnp.zeros((16,), jnp.float32)
  eps16 = jnp.full((16,), 1e-06, jnp.float32)

  def body(wv_acc, z_acc):
    def _init(i, _):
      for j in range(8):
        sl = pl.ds(16 * j, 16)
        wv_acc[i, sl] = zero16
        z_acc[i, sl] = eps16
      return 0

    lax.fori_loop(0, NPT + 1, _init, 0)

    pltpu.sync_copy(meta_h.at[w], meta_v)
    mv = meta_v[...]
    a_w = mv[0]
    e_w = mv[1]
    nch = lax.shift_right_logical(e_w - a_w + (CH - 1), 5)
    base_node = w * NPT

    def _chunk(ch, _):
      e0 = pl.multiple_of(a_w + ch * CH, 8)
      pltpu.sync_copy(srcs_h.at[pl.ds(e0, CH)], src_v)
      pltpu.sync_copy(dsts_h.at[pl.ds(e0, CH)], dst_v)
      pltpu.sync_copy(prm_h.at[pl.ds(e0, CH)], prm_v)
      cpk = pltpu.async_copy(k_tab.at[src_v], kb, sem)
      cpv = pltpu.async_copy(v_tab.at[src_v], vb, sem)
      cpq = pltpu.async_copy(q_tab.at[dst_v], qb, sem)
      cpp = pltpu.async_copy(pe_tab.at[prm_v], pb, sem)

      dls = []
      for g in range(CH // 16):
        dv = dst_v[pl.ds(16 * g, 16)] - base_node
        ok = (dv >= 0) & (dv < NPT)
        dls.append(jnp.where(ok, dv, DUMP))

      cpk.wait()
      cpv.wait()
      cpq.wait()
      cpp.wait()

      for g in range(CH // 16):
        dlv = dls[g]
        for e in range(16):
          i = g * 16 + e
          dle = dlv[e]
          for j in range(8):
            sl = pl.ds(16 * j, 16)
            kq = kb[i, sl] * qb[i, sl]
            a = _butterfly_sum(kq, perms) * quart
            sc = a * pb[i, sl]
            m = vb[i, sl] * sc
            z_acc[dle, sl] = z_acc[dle, sl] + sc
            wv_acc[dle, sl] = wv_acc[dle, sl] + m
      return 0

    lax.fori_loop(0, nch, _chunk, 0)

    ob = w * NPT
    pltpu.sync_copy(wv_acc.at[pl.ds(0, NPT)], wv_out.at[pl.ds(ob, NPT)])
    pltpu.sync_copy(z_acc.at[pl.ds(0, NPT)], z_out.at[pl.ds(ob, NPT)])

  pl.run_scoped(
      body,
      wv_acc=pltpu.VMEM((NPT + 8, D), jnp.float32),
      z_acc=pltpu.VMEM((NPT + 8, D), jnp.float32),
  )


def _edge_stage(q_tab, k_tab, v_tab, pe_tab, srcs, dsts, prm, meta):
  mesh = plsc.VectorSubcoreMesh(core_axis_name="c", subcore_axis_name="s",
                                num_cores=NC, num_subcores=NS)
  f32 = jnp.float32
  i32 = jnp.int32
  kfn = pl.kernel(
      _edge_kernel,
      out_type=(
          jax.ShapeDtypeStruct((NPAD, D), f32),
          jax.ShapeDtypeStruct((NPAD, D), f32),
      ),
      mesh=mesh,
      compiler_params=pltpu.CompilerParams(needs_layout_passes=False,
                                           use_tc_tiling_on_sc=False),
      scratch_types=[
          pltpu.VMEM((CH,), i32),
          pltpu.VMEM((CH,), i32),
          pltpu.VMEM((CH,), i32),
          pltpu.VMEM((16,), i32),
          pltpu.VMEM((CH, D), f32),
          pltpu.VMEM((CH, D), f32),
          pltpu.VMEM((CH, D), f32),
          pltpu.VMEM((CH, D), f32),
          pltpu.SemaphoreType.DMA,
      ],
  )
  return kfn(q_tab, k_tab, v_tab, pe_tab, srcs, dsts, prm, meta)


def kernel(feat, position, edge_index, W_Q, b_Q, W_K, b_K, W_V, b_V,
           W_e, b_e, W_s, b_s):
  src = edge_index[0]
  dst = edge_index[1]

  Q = feat @ W_Q + b_Q
  K = feat @ W_K + b_K
  V = feat @ W_V + b_V
  h_self = feat @ W_s + b_s

  rel = position[dst] - position[src]
  spatial_scale = jnp.linalg.norm(rel, axis=1) + EPS
  spatial_att = rel + 1.0
  e = spatial_att / spatial_scale[:, None]
  pe = e @ W_e + b_e

  bucket = dst // NPT
  order = jnp.argsort(bucket, stable=True).astype(jnp.int32)
  srcs = src[order]
  dsts = dst[order]
  bsort = bucket[order]
  starts = jnp.searchsorted(bsort, jnp.arange(NW + 1, dtype=jnp.int32),
                            side="left").astype(jnp.int32)
  astart = starts[:NW] & ~jnp.int32(7)
  meta = jnp.stack([astart, starts[1:]], axis=1)
  meta = jnp.pad(meta, ((0, 0), (0, 14)))

  padn = EPADDED - E
  srcs = jnp.concatenate([srcs, jnp.zeros((padn,), jnp.int32)])
  dsts = jnp.concatenate([dsts, jnp.full((padn,), NPAD + NPT, jnp.int32)])
  prm = jnp.concatenate([order, jnp.zeros((padn,), jnp.int32)])

  wv, z = _edge_stage(Q, K, V, pe, srcs, dsts, prm, meta)

  wv_h = wv[:N].reshape(N, NUM_HEADS, OUT_DIM)
  z_h = z[:N].reshape(N, NUM_HEADS, OUT_DIM)
  h_out = h_self.reshape(N, NUM_HEADS, OUT_DIM) + wv_h / z_h
  return h_out


# speedup vs baseline: 13.5365x; 1.0949x over previous
import functools

import jax
import jax.numpy as jnp
import numpy as np
from jax import lax
from jax.experimental import pallas as pl
from jax.experimental.pallas import tpu as pltpu
from jax.experimental.pallas import tpu_sc as plsc

N = 10000
E = 320000
IN_DIM = 128
OUT_DIM = 16
NUM_HEADS = 8
D = OUT_DIM * NUM_HEADS
EPS = 1e-07

NC = 2
NS = 16
NW = NC * NS
NPT = 320
NPAD = NW * NPT
DUMP = NPT
CH = 32
EPADDED = E + 2 * CH


def _butterfly_sum(x, perms):
  for p in perms:
    x = x + jnp.take_along_axis(x, p, axis=0, mode="promise_in_bounds")
  return x


def _edge_kernel(q_tab, k_tab, v_tab, pe_tab, srcs_h, dsts_h, prm_h, meta_h,
                 wv_out, z_out,
                 src_v, dst_v, prm_v, meta_v, kb, qb, vb, pb, sem):
  w = lax.axis_index("c") * NS + lax.axis_index("s")

  lanes = lax.iota(jnp.int32, 16)
  perms = [lanes ^ k for k in (1, 2, 4, 8)]
  quart = jnp.full((16,), 0.25, jnp.float32)
  zero16 = jnp.zeros((16,), jnp.float32)
  eps16 = jnp.full((16,), 1e-06, jnp.float32)

  def body(wv_acc, z_acc):
    def _init(i, _):
      for j in range(8):
        sl = pl.ds(16 * j, 16)
        wv_acc[i, sl] = zero16
        z_acc[i, sl] = eps16
      return 0

    lax.fori_loop(0, NPT + 1, _init, 0)

    pltpu.sync_copy(meta_h.at[w], meta_v)
    mv = meta_v[...]
    a_w = mv[0]
    e_w = mv[1]
    nch = lax.shift_right_logical(e_w - a_w + (CH - 1), 5)
    base_node = w * NPT

    def _chunk(ch, _):
      e0 = pl.multiple_of(a_w + ch * CH, 8)
      pltpu.sync_copy(srcs_h.at[pl.ds(e0, CH)], src_v)
      pltpu.sync_copy(dsts_h.at[pl.ds(e0, CH)], dst_v)
      pltpu.sync_copy(prm_h.at[pl.ds(e0, CH)], prm_v)
      cpk = pltpu.async_copy(k_tab.at[src_v], kb, sem)
      cpv = pltpu.async_copy(v_tab.at[src_v], vb, sem)
      cpq = pltpu.async_copy(q_tab.at[dst_v], qb, sem)
      cpp = pltpu.async_copy(pe_tab.at[prm_v], pb, sem)

      dls = []
      for g in range(CH // 16):
        dv = dst_v[pl.ds(16 * g, 16)] - base_node
        ok = (dv >= 0) & (dv < NPT)
        dls.append(jnp.where(ok, dv, DUMP))

      cpk.wait()
      cpv.wait()
      cpq.wait()
      cpp.wait()

      for g in range(CH // 16):
        dlv = dls[g]
        for e in range(16):
          i = g * 16 + e
          dle = dlv[e]
          for j in range(8):
            sl = pl.ds(16 * j, 16)
            kq = kb[i, sl] * qb[i, sl]
            a = _butterfly_sum(kq, perms) * quart
            sc = a * pb[i, sl]
            m = vb[i, sl] * sc
            plsc.addupdate(z_acc.at[dle, sl], sc)
            plsc.addupdate(wv_acc.at[dle, sl], m)
      return 0

    lax.fori_loop(0, nch, _chunk, 0)

    ob = w * NPT
    pltpu.sync_copy(wv_acc.at[pl.ds(0, NPT)], wv_out.at[pl.ds(ob, NPT)])
    pltpu.sync_copy(z_acc.at[pl.ds(0, NPT)], z_out.at[pl.ds(ob, NPT)])

  pl.run_scoped(
      body,
      wv_acc=pltpu.VMEM((NPT + 8, D), jnp.float32),
      z_acc=pltpu.VMEM((NPT + 8, D), jnp.float32),
  )


def _edge_stage(q_tab, k_tab, v_tab, pe_tab, srcs, dsts, prm, meta):
  mesh = plsc.VectorSubcoreMesh(core_axis_name="c", subcore_axis_name="s",
                                num_cores=NC, num_subcores=NS)
  f32 = jnp.float32
  i32 = jnp.int32
  kfn = pl.kernel(
      _edge_kernel,
      out_type=(
          jax.ShapeDtypeStruct((NPAD, D), f32),
          jax.ShapeDtypeStruct((NPAD, D), f32),
      ),
      mesh=mesh,
      compiler_params=pltpu.CompilerParams(needs_layout_passes=False,
                                           use_tc_tiling_on_sc=False),
      scratch_types=[
          pltpu.VMEM((CH,), i32),
          pltpu.VMEM((CH,), i32),
          pltpu.VMEM((CH,), i32),
          pltpu.VMEM((16,), i32),
          pltpu.VMEM((CH, D), f32),
          pltpu.VMEM((CH, D), f32),
          pltpu.VMEM((CH, D), f32),
          pltpu.VMEM((CH, D), f32),
          pltpu.SemaphoreType.DMA,
      ],
  )
  return kfn(q_tab, k_tab, v_tab, pe_tab, srcs, dsts, prm, meta)


def kernel(feat, position, edge_index, W_Q, b_Q, W_K, b_K, W_V, b_V,
           W_e, b_e, W_s, b_s):
  src = edge_index[0]
  dst = edge_index[1]

  Q = feat @ W_Q + b_Q
  K = feat @ W_K + b_K
  V = feat @ W_V + b_V
  h_self = feat @ W_s + b_s

  rel = position[dst] - position[src]
  spatial_scale = jnp.linalg.norm(rel, axis=1) + EPS
  spatial_att = rel + 1.0
  e = spatial_att / spatial_scale[:, None]
  pe = e @ W_e + b_e

  bucket = dst // NPT
  order = jnp.argsort(bucket, stable=True).astype(jnp.int32)
  srcs = src[order]
  dsts = dst[order]
  bsort = bucket[order]
  starts = jnp.searchsorted(bsort, jnp.arange(NW + 1, dtype=jnp.int32),
                            side="left").astype(jnp.int32)
  astart = starts[:NW] & ~jnp.int32(7)
  meta = jnp.stack([astart, starts[1:]], axis=1)
  meta = jnp.pad(meta, ((0, 0), (0, 14)))

  padn = EPADDED - E
  srcs = jnp.concatenate([srcs, jnp.zeros((padn,), jnp.int32)])
  dsts = jnp.concatenate([dsts, jnp.full((padn,), NPAD + NPT, jnp.int32)])
  prm = jnp.concatenate([order, jnp.zeros((padn,), jnp.int32)])

  wv, z = _edge_stage(Q, K, V, pe, srcs, dsts, prm, meta)

  wv_h = wv[:N].reshape(N, NUM_HEADS, OUT_DIM)
  z_h = z[:N].reshape(N, NUM_HEADS, OUT_DIM)
  h_out = h_self.reshape(N, NUM_HEADS, OUT_DIM) + wv_h / z_h
  return h_out


# speedup vs baseline: 16.0368x; 1.1847x over previous
import functools

import jax
import jax.numpy as jnp
import numpy as np
from jax import lax
from jax.experimental import pallas as pl
from jax.experimental.pallas import tpu as pltpu
from jax.experimental.pallas import tpu_sc as plsc

N = 10000
E = 320000
IN_DIM = 128
OUT_DIM = 16
NUM_HEADS = 8
D = OUT_DIM * NUM_HEADS
EPS = 1e-07

NC = 2
NS = 16
NW = NC * NS
NPT = 320
NPAD = NW * NPT
DUMP = NPT
CH = 32
EPADDED = E + 2 * CH
NB = EPADDED // CH


def _butterfly_sum(x, perms):
  for p in perms:
    x = x + jnp.take_along_axis(x, p, axis=0, mode="promise_in_bounds")
  return x


def _edge_kernel(kv_tab, q_tab, pe_tab, ed_h, meta_h,
                 wv_out, z_out,
                 ib, kvb, qb, pb, meta_v, semi, semg):
  w = lax.axis_index("c") * NS + lax.axis_index("s")

  lanes = lax.iota(jnp.int32, 16)
  perms = [lanes ^ k for k in (1, 2, 4, 8)]
  quart = jnp.full((16,), 0.25, jnp.float32)
  zero16 = jnp.zeros((16,), jnp.float32)
  eps16 = jnp.full((16,), 1e-06, jnp.float32)

  def body(wv_acc, z_acc):
    def _init(i, _):
      for j in range(8):
        sl = pl.ds(16 * j, 16)
        wv_acc[i, sl] = zero16
        z_acc[i, sl] = eps16
      return 0

    lax.fori_loop(0, NPT + 1, _init, 0)

    pltpu.sync_copy(meta_h.at[w], meta_v)
    mv = meta_v[...]
    a_w = mv[0]
    e_w = mv[1]
    nch = lax.shift_right_logical(e_w - a_w + (CH - 1), 5)
    b_w = lax.shift_right_logical(pl.multiple_of(a_w, CH), 5)
    base_node = w * NPT

    def _issue_idx(slot, b):
      return pltpu.async_copy(ed_h.at[b], ib.at[slot], semi.at[slot])

    def _wait_idx(slot):
      pltpu.make_async_copy(ed_h.at[0], ib.at[slot], semi.at[slot]).wait()

    def _issue_gathers(slot):
      pltpu.async_copy(kv_tab.at[ib.at[slot, 0]], kvb.at[slot],
                       semg.at[slot])
      pltpu.async_copy(q_tab.at[ib.at[slot, 1]], qb.at[slot], semg.at[slot])
      pltpu.async_copy(pe_tab.at[ib.at[slot, 2]], pb.at[slot],
                       semg.at[slot])

    def _wait_gathers(slot):
      pltpu.make_async_copy(kv_tab.at[pl.ds(0, CH)], kvb.at[slot],
                            semg.at[slot]).wait()
      pltpu.make_async_copy(q_tab.at[pl.ds(0, CH)], qb.at[slot],
                            semg.at[slot]).wait()
      pltpu.make_async_copy(pe_tab.at[pl.ds(0, CH)], pb.at[slot],
                            semg.at[slot]).wait()

    @pl.when(nch >= 1)
    def _prologue():
      _issue_idx(0, b_w).wait()
      _issue_gathers(0)

      @pl.when(nch >= 2)
      def _():
        _issue_idx(1, b_w + 1)

    def _chunk(ch, _):
      p = ch & 1
      q = 1 - p
      _wait_gathers(p)

      dls = []
      for g in range(CH // 16):
        dv = ib[p, 1, pl.ds(16 * g, 16)] - base_node
        ok = (dv >= 0) & (dv < NPT)
        dls.append(jnp.where(ok, dv, DUMP))

      @pl.when(ch + 1 < nch)
      def _():
        _wait_idx(q)
        _issue_gathers(q)

      @pl.when(ch + 2 < nch)
      def _():
        _issue_idx(p, b_w + ch + 2)

      for g in range(CH // 16):
        dlv = dls[g]
        for e in range(16):
          i = g * 16 + e
          dle = dlv[e]
          for j in range(8):
            sl = pl.ds(16 * j, 16)
            kq = kvb[p, i, sl] * qb[p, i, sl]
            a = _butterfly_sum(kq, perms) * quart
            sc = a * pb[p, i, sl]
            m = kvb[p, i, pl.ds(D + 16 * j, 16)] * sc
            plsc.addupdate(z_acc.at[dle, sl], sc)
            plsc.addupdate(wv_acc.at[dle, sl], m)
      return 0

    lax.fori_loop(0, nch, _chunk, 0)

    ob = w * NPT
    pltpu.sync_copy(wv_acc.at[pl.ds(0, NPT)], wv_out.at[pl.ds(ob, NPT)])
    pltpu.sync_copy(z_acc.at[pl.ds(0, NPT)], z_out.at[pl.ds(ob, NPT)])

  pl.run_scoped(
      body,
      wv_acc=pltpu.VMEM((NPT + 8, D), jnp.float32),
      z_acc=pltpu.VMEM((NPT + 8, D), jnp.float32),
  )


def _edge_stage(kv_tab, q_tab, pe_tab, ed, meta):
  mesh = plsc.VectorSubcoreMesh(core_axis_name="c", subcore_axis_name="s",
                                num_cores=NC, num_subcores=NS)
  f32 = jnp.float32
  i32 = jnp.int32
  kfn = pl.kernel(
      _edge_kernel,
      out_type=(
          jax.ShapeDtypeStruct((NPAD, D), f32),
          jax.ShapeDtypeStruct((NPAD, D), f32),
      ),
      mesh=mesh,
      compiler_params=pltpu.CompilerParams(needs_layout_passes=False,
                                           use_tc_tiling_on_sc=False),
      scratch_types=[
          pltpu.VMEM((2, 3, CH), i32),
          pltpu.VMEM((2, CH, 2 * D), f32),
          pltpu.VMEM((2, CH, D), f32),
          pltpu.VMEM((2, CH, D), f32),
          pltpu.VMEM((16,), i32),
          pltpu.SemaphoreType.DMA((2,)),
          pltpu.SemaphoreType.DMA((2,)),
      ],
  )
  return kfn(kv_tab, q_tab, pe_tab, ed, meta)


def kernel(feat, position, edge_index, W_Q, b_Q, W_K, b_K, W_V, b_V,
           W_e, b_e, W_s, b_s):
  src = edge_index[0]
  dst = edge_index[1]

  Q = feat @ W_Q + b_Q
  K = feat @ W_K + b_K
  V = feat @ W_V + b_V
  h_self = feat @ W_s + b_s
  KV = jnp.concatenate([K, V], axis=1)

  rel = position[dst] - position[src]
  spatial_scale = jnp.linalg.norm(rel, axis=1) + EPS
  spatial_att = rel + 1.0
  e = spatial_att / spatial_scale[:, None]
  pe = e @ W_e + b_e

  bucket = dst // NPT
  order = jnp.argsort(bucket, stable=True).astype(jnp.int32)
  srcs = src[order]
  dsts = dst[order]
  bsort = bucket[order]
  starts = jnp.searchsorted(bsort, jnp.arange(NW + 1, dtype=jnp.int32),
                            side="left").astype(jnp.int32)
  astart = starts[:NW] & ~jnp.int32(CH - 1)
  meta = jnp.stack([astart, starts[1:]], axis=1)
  meta = jnp.pad(meta, ((0, 0), (0, 14)))

  padn = EPADDED - E
  srcs = jnp.concatenate([srcs, jnp.zeros((padn,), jnp.int32)])
  dsts = jnp.concatenate([dsts, jnp.full((padn,), NPAD + NPT, jnp.int32)])
  prm = jnp.concatenate([order, jnp.zeros((padn,), jnp.int32)])
  ed = jnp.stack([srcs.reshape(NB, CH), dsts.reshape(NB, CH),
                  prm.reshape(NB, CH)], axis=1)

  wv, z = _edge_stage(KV, Q, pe, ed, meta)

  wv_h = wv[:N].reshape(N, NUM_HEADS, OUT_DIM)
  z_h = z[:N].reshape(N, NUM_HEADS, OUT_DIM)
  h_out = h_self.reshape(N, NUM_HEADS, OUT_DIM) + wv_h / z_h
  return h_out


# speedup vs baseline: 24.8929x; 1.5522x over previous
import functools

import jax
import jax.numpy as jnp
import numpy as np
from jax import lax
from jax.experimental import pallas as pl
from jax.experimental.pallas import tpu as pltpu
from jax.experimental.pallas import tpu_sc as plsc

N = 10000
E = 320000
IN_DIM = 128
OUT_DIM = 16
NUM_HEADS = 8
D = OUT_DIM * NUM_HEADS
EPS = 1e-07

NC = 2
NS = 16
NW = NC * NS
NPT = 320
NPAD = NW * NPT
DUMP = NPT
CH = 32
EPADDED = E + 2 * CH
NB = EPADDED // CH


def _butterfly_sum(x, perms):
  for p in perms:
    x = x + jnp.take_along_axis(x, p, axis=0, mode="promise_in_bounds")
  return x


def _edge_kernel(kv_tab, q_tab, pe_tab, ed_h, meta_h,
                 wv_out, z_out,
                 ib, kvb, qb, pb, meta_v, semi, semg):
  w = lax.axis_index("c") * NS + lax.axis_index("s")

  lanes = lax.iota(jnp.int32, 16)
  perms = [lanes ^ k for k in (1, 2, 4, 8)]
  quart = jnp.full((16,), 0.25, jnp.float32)
  zero16 = jnp.zeros((16,), jnp.float32)
  eps16 = jnp.full((16,), 1e-06, jnp.float32)

  def body(wv_acc, z_acc):
    def _init(i, _):
      for j in range(8):
        sl = pl.ds(16 * j, 16)
        wv_acc[pl.ds(i * D + 16 * j, 16)] = zero16
        z_acc[pl.ds(i * D + 16 * j, 16)] = eps16
      return 0

    lax.fori_loop(0, NPT + 1, _init, 0)

    pltpu.sync_copy(meta_h.at[w], meta_v)
    mv = meta_v[...]
    a_w = mv[0]
    e_w = mv[1]
    nch = lax.shift_right_logical(e_w - a_w + (CH - 1), 5)
    b_w = lax.shift_right_logical(pl.multiple_of(a_w, CH), 5)
    base_node = w * NPT

    def _issue_idx(slot, b):
      return pltpu.async_copy(ed_h.at[b], ib.at[slot], semi.at[slot])

    def _wait_idx(slot):
      pltpu.make_async_copy(ed_h.at[0], ib.at[slot], semi.at[slot]).wait()

    def _issue_gathers(slot):
      pltpu.async_copy(kv_tab.at[ib.at[slot, 0]], kvb.at[slot],
                       semg.at[slot])
      pltpu.async_copy(q_tab.at[ib.at[slot, 1]], qb.at[slot], semg.at[slot])
      pltpu.async_copy(pe_tab.at[ib.at[slot, 2]], pb.at[slot],
                       semg.at[slot])

    def _wait_gathers(slot):
      pltpu.make_async_copy(kv_tab.at[pl.ds(0, CH)], kvb.at[slot],
                            semg.at[slot]).wait()
      pltpu.make_async_copy(q_tab.at[pl.ds(0, CH)], qb.at[slot],
                            semg.at[slot]).wait()
      pltpu.make_async_copy(pe_tab.at[pl.ds(0, CH)], pb.at[slot],
                            semg.at[slot]).wait()

    @pl.when(nch >= 1)
    def _prologue():
      _issue_idx(0, b_w).wait()
      _issue_gathers(0)

      @pl.when(nch >= 2)
      def _():
        _issue_idx(1, b_w + 1)

    def _chunk(ch, _):
      p = ch & 1
      q = 1 - p
      _wait_gathers(p)

      dvs = [ib[p, 1, pl.ds(16 * g, 16)] for g in range(CH // 16)]
      base_vec = jnp.full((16,), 0, jnp.int32) + base_node

      @pl.when(ch + 1 < nch)
      def _():
        _wait_idx(q)
        _issue_gathers(q)

      @pl.when(ch + 2 < nch)
      def _():
        _issue_idx(p, b_w + ch + 2)

      for g in range(CH // 16):
        dvv = dvs[g]
        for e in range(16):
          i = g * 16 + e
          d0 = jnp.take_along_axis(dvv, lanes * 0 + e, axis=0,
                                   mode="promise_in_bounds") - base_vec
          ok = jnp.logical_and(d0 >= 0, d0 < NPT)
          dl = jnp.where(ok, d0, DUMP)
          fb = dl * D + lanes
          accs = []
          for j in range(8):
            sl = pl.ds(16 * j, 16)
            idx = fb + 16 * j
            kq = kvb[p, i, sl] * qb[p, i, sl]
            a = _butterfly_sum(kq, perms) * quart
            sc = a * pb[p, i, sl]
            m = kvb[p, i, pl.ds(D + 16 * j, 16)] * sc
            accs.append((idx, sc, m))
          for idx, sc, m in accs:
            plsc.addupdate_scatter(z_acc, [idx], sc)
            plsc.addupdate_scatter(wv_acc, [idx], m)
      return 0

    lax.fori_loop(0, nch, _chunk, 0)

    ob = w * NPT * D
    pltpu.sync_copy(wv_acc.at[pl.ds(0, NPT * D)], wv_out.at[pl.ds(ob, NPT * D)])
    pltpu.sync_copy(z_acc.at[pl.ds(0, NPT * D)], z_out.at[pl.ds(ob, NPT * D)])

  pl.run_scoped(
      body,
      wv_acc=pltpu.VMEM(((NPT + 8) * D,), jnp.float32),
      z_acc=pltpu.VMEM(((NPT + 8) * D,), jnp.float32),
  )


def _edge_stage(kv_tab, q_tab, pe_tab, ed, meta):
  mesh = plsc.VectorSubcoreMesh(core_axis_name="c", subcore_axis_name="s",
                                num_cores=NC, num_subcores=NS)
  f32 = jnp.float32
  i32 = jnp.int32
  kfn = pl.kernel(
      _edge_kernel,
      out_type=(
          jax.ShapeDtypeStruct((NPAD * D,), f32),
          jax.ShapeDtypeStruct((NPAD * D,), f32),
      ),
      mesh=mesh,
      compiler_params=pltpu.CompilerParams(needs_layout_passes=False,
                                           use_tc_tiling_on_sc=False),
      scratch_types=[
          pltpu.VMEM((2, 3, CH), i32),
          pltpu.VMEM((2, CH, 2 * D), f32),
          pltpu.VMEM((2, CH, D), f32),
          pltpu.VMEM((2, CH, D), f32),
          pltpu.VMEM((16,), i32),
          pltpu.SemaphoreType.DMA((2,)),
          pltpu.SemaphoreType.DMA((2,)),
      ],
  )
  return kfn(kv_tab, q_tab, pe_tab, ed, meta)


def kernel(feat, position, edge_index, W_Q, b_Q, W_K, b_K, W_V, b_V,
           W_e, b_e, W_s, b_s):
  src = edge_index[0]
  dst = edge_index[1]

  Q = feat @ W_Q + b_Q
  K = feat @ W_K + b_K
  V = feat @ W_V + b_V
  h_self = feat @ W_s + b_s
  KV = jnp.concatenate([K, V], axis=1)

  rel = position[dst] - position[src]
  spatial_scale = jnp.linalg.norm(rel, axis=1) + EPS
  spatial_att = rel + 1.0
  e = spatial_att / spatial_scale[:, None]
  pe = e @ W_e + b_e

  bucket = dst // NPT
  order = jnp.argsort(bucket, stable=True).astype(jnp.int32)
  srcs = src[order]
  dsts = dst[order]
  bsort = bucket[order]
  starts = jnp.searchsorted(bsort, jnp.arange(NW + 1, dtype=jnp.int32),
                            side="left").astype(jnp.int32)
  astart = starts[:NW] & ~jnp.int32(CH - 1)
  meta = jnp.stack([astart, starts[1:]], axis=1)
  meta = jnp.pad(meta, ((0, 0), (0, 14)))

  padn = EPADDED - E
  srcs = jnp.concatenate([srcs, jnp.zeros((padn,), jnp.int32)])
  dsts = jnp.concatenate([dsts, jnp.full((padn,), NPAD + NPT, jnp.int32)])
  prm = jnp.concatenate([order, jnp.zeros((padn,), jnp.int32)])
  ed = jnp.stack([srcs.reshape(NB, CH), dsts.reshape(NB, CH),
                  prm.reshape(NB, CH)], axis=1)

  wv, z = _edge_stage(KV, Q, pe, ed, meta)

  wv_h = wv[:N * D].reshape(N, NUM_HEADS, OUT_DIM)
  z_h = z[:N * D].reshape(N, NUM_HEADS, OUT_DIM)
  h_out = h_self.reshape(N, NUM_HEADS, OUT_DIM) + wv_h / z_h
  return h_out
